# baseline (speedup 1.0000x reference)
import functools
import math

import jax
import jax.numpy as jnp
from jax import lax
from jax.experimental import pallas as pl
from jax.experimental.pallas import tpu as pltpu
from jax.experimental.pallas import tpu_sc as plsc

N_ANCHORS = 20000
NPAD = 32768
C = 128
R = NPAD // C
K2 = 2048
KR = K2 // C
PRE_NMS = 2000
POST_NMS = 1000
OPAD = 1024
NMS_THRESH = 0.7
MIN_SIZE = 1e-3
IMG = 800.0
BBOX_CLIP = float(math.log(1000.0 / 16.0))
TBL_W = 16


def _xshuf(x, d):
    if d < C:
        bit = (lax.broadcasted_iota(jnp.int32, x.shape, 1) & d) != 0
        return jnp.where(bit, pltpu.roll(x, d, axis=1),
                         pltpu.roll(x, C - d, axis=1))
    m = d // C
    bit = (lax.broadcasted_iota(jnp.int32, x.shape, 0) & m) != 0
    return jnp.where(bit, pltpu.roll(x, m, axis=0),
                     pltpu.roll(x, x.shape[0] - m, axis=0))


def _cmpex(key, idx, pos, p, q):
    d = 1 << q
    pk = _xshuf(key, d)
    pi = _xshuf(idx, d)
    self_first = (key > pk) | ((key == pk) & (idx < pi))
    run_up = (pos & (2 << p)) != 0
    hi_half = (pos & d) != 0
    take = self_first ^ run_up ^ hi_half
    return jnp.where(take, key, pk), jnp.where(take, idx, pi)


def _posmat(rows):
    return (lax.broadcasted_iota(jnp.int32, (rows, C), 0) * C
            + lax.broadcasted_iota(jnp.int32, (rows, C), 1))


def _sort_body(obj_ref, idx_out, lidx_out, key_out):
    nb = obj_ref.shape[0]
    pos = _posmat(R)
    keys = [obj_ref[b] for b in range(nb)]
    idxs = [pos] * nb
    for p in range(11):
        for q in range(p, -1, -1):
            for b in range(nb):
                keys[b], idxs[b] = _cmpex(keys[b], idxs[b], pos, p, q)
    rows = R
    while rows > KR:
        g = rows // 32
        nrows = rows // 2
        npos = _posmat(nrows)
        for b in range(nb):
            k3 = keys[b].reshape(g, 32, C)
            i3 = idxs[b].reshape(g, 32, C)
            ka, kb = k3[:, :16], k3[:, 16:]
            ia, ib = i3[:, :16], i3[:, 16:]
            sf = (ka > kb) | ((ka == kb) & (ia < ib))
            keys[b] = jnp.where(sf, ka, kb).reshape(nrows, C)
            idxs[b] = jnp.where(sf, ia, ib).reshape(nrows, C)
        rows = nrows
        for q in range(10, -1, -1):
            for b in range(nb):
                keys[b], idxs[b] = _cmpex(keys[b], idxs[b], npos, 10, q)
    for b in range(nb):
        idx_out[b] = idxs[b] + b * N_ANCHORS
        lidx_out[b] = idxs[b]
        key_out[b] = keys[b]


def _topk_sort(obj3):
    b = obj3.shape[0]
    return pl.pallas_call(
        _sort_body,
        out_shape=[jax.ShapeDtypeStruct((b, KR, C), jnp.int32),
                   jax.ShapeDtypeStruct((b, KR, C), jnp.int32),
                   jax.ShapeDtypeStruct((b, KR, C), jnp.float32)],
    )(obj3)


def _sc_gather(dtbl, atbl, gidx, lidx):
    nrows = gidx.shape[0]
    info = plsc.get_sparse_core_info()
    nw = info.num_cores * info.num_subcores
    per = nrows // nw
    mesh = plsc.VectorSubcoreMesh(core_axis_name="c", subcore_axis_name="s")

    @functools.partial(
        pl.kernel,
        out_type=[jax.ShapeDtypeStruct((nrows, 4), jnp.float32),
                  jax.ShapeDtypeStruct((nrows, 4), jnp.float32)],
        mesh=mesh,
        scratch_types=[pltpu.VMEM((per,), jnp.int32),
                       pltpu.VMEM((per,), jnp.int32),
                       pltpu.VMEM((per, 4), jnp.float32),
                       pltpu.VMEM((per, 4), jnp.float32),
                       pltpu.SemaphoreType.DMA,
                       pltpu.SemaphoreType.DMA],
        compiler_params=pltpu.CompilerParams(use_tc_tiling_on_sc=False),
    )
    def k(dtbl_hbm, atbl_hbm, gidx_hbm, lidx_hbm, dout_hbm, aout_hbm,
          gidx_v, lidx_v, drows_v, arows_v, dsem, asem):
        wid = lax.axis_index("s") * info.num_cores + lax.axis_index("c")
        base = wid * per
        pltpu.sync_copy(gidx_hbm.at[pl.ds(base, per)], gidx_v)
        pltpu.sync_copy(lidx_hbm.at[pl.ds(base, per)], lidx_v)
        dcp = pltpu.async_copy(dtbl_hbm.at[gidx_v], drows_v, dsem)
        acp = pltpu.async_copy(atbl_hbm.at[lidx_v], arows_v, asem)
        dcp.wait()
        acp.wait()
        pltpu.sync_copy(drows_v, dout_hbm.at[pl.ds(base, per)])
        pltpu.sync_copy(arows_v, aout_hbm.at[pl.ds(base, per)])

    return k(dtbl, atbl, gidx, lidx)


def _decode(dx, dy, dw, dh, ax1, ay1, ax2, ay2):
    w = ax2 - ax1
    h = ay2 - ay1
    cx = ax1 + 0.5 * w
    cy = ay1 + 0.5 * h
    dwc = jnp.minimum(dw, BBOX_CLIP)
    dhc = jnp.minimum(dh, BBOX_CLIP)
    pcx = dx * w + cx
    pcy = dy * h + cy
    pw = jnp.exp(dwc) * w
    ph = jnp.exp(dhc) * h
    x1 = jnp.clip(pcx - 0.5 * pw, 0.0, IMG)
    y1 = jnp.clip(pcy - 0.5 * ph, 0.0, IMG)
    x2 = jnp.clip(pcx + 0.5 * pw, 0.0, IMG)
    y2 = jnp.clip(pcy + 0.5 * ph, 0.0, IMG)
    return x1, y1, x2, y2


RB = 256


def _build_m(b, msc, x1c, y1c, x2c, y2c, x1r, y1r, x2r, y2r,
             area_c, area_r, posc, posr, finite_c):
    for rb in range(K2 // RB):
        lo = rb * RB
        ltx = jnp.maximum(x1c[lo:lo + RB], x1r[:, lo:])
        lty = jnp.maximum(y1c[lo:lo + RB], y1r[:, lo:])
        rbx = jnp.minimum(x2c[lo:lo + RB], x2r[:, lo:])
        rby = jnp.minimum(y2c[lo:lo + RB], y2r[:, lo:])
        iw = jnp.maximum(rbx - ltx, 0.0)
        ih = jnp.maximum(rby - lty, 0.0)
        inter = iw * ih
        union = area_c[lo:lo + RB] + area_r[:, lo:] - inter
        iou = inter / jnp.maximum(union, 1e-9)
        m = ((iou > NMS_THRESH) & (posc[lo:lo + RB] < posr[:, lo:])
             & finite_c[lo:lo + RB])
        msc[b, lo:lo + RB, lo:] = m.astype(jnp.bfloat16)


def _nms_body(keyr_ref, gd_ref, ga_ref,
              bo, so,
              msc):
    nb = keyr_ref.shape[0]
    posr = lax.broadcasted_iota(jnp.int32, (1, K2), 1)
    posc = lax.broadcasted_iota(jnp.int32, (K2, 1), 0)
    rowsd, finrs, keyrs = [], [], []
    for b in range(nb):
        keyr = keyr_ref[b]
        gd = gd_ref[b * K2:(b + 1) * K2, :]
        ga = ga_ref[b * K2:(b + 1) * K2, :]
        gdt = jnp.transpose(gd)
        gat = jnp.transpose(ga)
        cf = ([gd[:, c:c + 1] for c in range(4)]
              + [ga[:, c:c + 1] for c in range(4)])
        rf = ([gdt[c:c + 1, :] for c in range(4)]
              + [gat[c:c + 1, :] for c in range(4)])
        x1r, y1r, x2r, y2r = _decode(*rf)
        x1c, y1c, x2c, y2c = _decode(*cf)
        finite_r = ((x2r - x1r >= MIN_SIZE) & (y2r - y1r >= MIN_SIZE)
                    & (posr < PRE_NMS))
        finite_c = ((x2c - x1c >= MIN_SIZE) & (y2c - y1c >= MIN_SIZE)
                    & (posc < PRE_NMS))
        area_r = jnp.maximum(x2r - x1r, 0.0) * jnp.maximum(y2r - y1r, 0.0)
        area_c = jnp.maximum(x2c - x1c, 0.0) * jnp.maximum(y2c - y1c, 0.0)
        _build_m(b, msc, x1c, y1c, x2c, y2c, x1r, y1r, x2r, y2r,
                 area_c, area_r, posc, posr, finite_c)
        rowsd.append((x1r, y1r, x2r, y2r))
        finrs.append(finite_r)
        keyrs.append(keyr)

    keep0 = jnp.concatenate(
        [jnp.broadcast_to(f.astype(jnp.float32), (8, K2)) for f in finrs],
        axis=0)

    def cond(st):
        it, changed, _ = st
        return changed & (it < K2)

    def body(st):
        it, _, keep = st
        kb = keep.astype(jnp.bfloat16)
        sups = []
        for b in range(nb):
            kbb = kb[8 * b:8 * b + 8]
            sups.append(jnp.concatenate(
                [jnp.dot(kbb[:, :cb * RB + RB],
                         msc[b, 0:cb * RB + RB, cb * RB:cb * RB + RB],
                         preferred_element_type=jnp.float32)
                 for cb in range(K2 // RB)], axis=1))
        sup = jnp.concatenate(sups, axis=0)
        keepn = jnp.where(sup > 0.5, 0.0, keep0)
        return it + 1, jnp.any(keepn != keep), keepn

    _, _, keep = lax.while_loop(cond, body,
                                (jnp.int32(0), jnp.bool_(True), keep0))

    p256r = lax.broadcasted_iota(jnp.int32, (1, RB), 1)
    p256c = lax.broadcasted_iota(jnp.int32, (RB, 1), 0)
    tri = (p256c <= p256r).astype(jnp.bfloat16)
    rr = lax.broadcasted_iota(jnp.int32, (OPAD, K2), 0) + 1
    for b in range(nb):
        kf = keep[8 * b:8 * b + 1]
        kb = keep[8 * b:8 * b + 8].astype(jnp.bfloat16)
        blocks = []
        off = jnp.float32(0.0)
        for cb in range(K2 // RB):
            lo = cb * RB
            rb_ = jnp.dot(kb[:, lo:lo + RB], tri,
                          preferred_element_type=jnp.float32) + off
            blocks.append(rb_)
            off = off + jnp.sum(kf[:, lo:lo + RB])
        rank = jnp.concatenate(blocks, axis=1)[0:1]
        rank_i = rank.astype(jnp.int32)
        P = ((rank_i == rr) & (kf > 0.5)).astype(jnp.float32)
        x1r, y1r, x2r, y2r = rowsd[b]
        score_r = jnp.where(finrs[b], keyrs[b], 0.0)
        bo[b] = jnp.concatenate(
            [jnp.sum(P * v, axis=1, keepdims=True)
             for v in (x1r, y1r, x2r, y2r)], axis=1)
        so[b] = jnp.sum(P * score_r, axis=1, keepdims=True)


def _nms_stage(keyr, gd, ga):
    b = keyr.shape[0]
    return pl.pallas_call(
        _nms_body,
        out_shape=[jax.ShapeDtypeStruct((b, OPAD, 4), jnp.float32),
                   jax.ShapeDtypeStruct((b, OPAD, 1), jnp.float32)],
        scratch_shapes=[pltpu.VMEM((b, K2, K2), jnp.bfloat16)],
        compiler_params=pltpu.CompilerParams(
            vmem_limit_bytes=120 * 1024 * 1024),
    )(keyr, gd, ga)


def kernel(objectness, pred_bbox_deltas, anchors):
    b = objectness.shape[0]
    n = anchors.shape[0]

    obj3 = jnp.full((b, NPAD), -jnp.inf, jnp.float32)
    obj3 = obj3.at[:, :n].set(objectness).reshape(b, R, C)
    sorted_idx, sorted_lidx, sorted_key = _topk_sort(obj3)

    gd, ga = _sc_gather(pred_bbox_deltas.reshape(b * n, 4), anchors,
                        sorted_idx.reshape(b * K2),
                        sorted_lidx.reshape(b * K2))

    keyr = sorted_key.reshape(b, 1, K2)
    bo, so = _nms_stage(keyr, gd, ga)

    boxes = bo[:, :POST_NMS, :]
    scores = so[:, :POST_NMS, 0]
    return boxes, scores


# speedup vs baseline: 1.3473x; 1.3473x over previous
import functools
import math

import jax
import jax.numpy as jnp
from jax import lax
from jax.experimental import pallas as pl
from jax.experimental.pallas import tpu as pltpu
from jax.experimental.pallas import tpu_sc as plsc

N_ANCHORS = 20000
NPAD = 32768
C = 128
R = NPAD // C
K2 = 2048
KR = K2 // C
PRE_NMS = 2000
POST_NMS = 1000
OPAD = 1024
NMS_THRESH = 0.7
MIN_SIZE = 1e-3
IMG = 800.0
BBOX_CLIP = float(math.log(1000.0 / 16.0))
TBL_W = 16


def _xshuf(x, d):
    if d < C:
        bit = (lax.broadcasted_iota(jnp.int32, x.shape, 1) & d) != 0
        return jnp.where(bit, pltpu.roll(x, d, axis=1),
                         pltpu.roll(x, C - d, axis=1))
    m = d // C
    bit = (lax.broadcasted_iota(jnp.int32, x.shape, 0) & m) != 0
    return jnp.where(bit, pltpu.roll(x, m, axis=0),
                     pltpu.roll(x, x.shape[0] - m, axis=0))


def _cmpex(key, idx, pos, p, q):
    d = 1 << q
    pk = _xshuf(key, d)
    pi = _xshuf(idx, d)
    self_first = (key > pk) | ((key == pk) & (idx < pi))
    run_up = (pos & (2 << p)) != 0
    hi_half = (pos & d) != 0
    take = self_first ^ run_up ^ hi_half
    return jnp.where(take, key, pk), jnp.where(take, idx, pi)


def _posmat(rows):
    return (lax.broadcasted_iota(jnp.int32, (rows, C), 0) * C
            + lax.broadcasted_iota(jnp.int32, (rows, C), 1))


def _sort_body(obj_ref, idx_out, lidx_out, key_out):
    nb = obj_ref.shape[0]
    pos = _posmat(R)
    keys = [obj_ref[b] for b in range(nb)]
    idxs = [pos] * nb
    for p in range(11):
        for q in range(p, -1, -1):
            for b in range(nb):
                keys[b], idxs[b] = _cmpex(keys[b], idxs[b], pos, p, q)
    rows = R
    while rows > KR:
        g = rows // 32
        nrows = rows // 2
        npos = _posmat(nrows)
        for b in range(nb):
            k3 = keys[b].reshape(g, 32, C)
            i3 = idxs[b].reshape(g, 32, C)
            ka, kb = k3[:, :16], k3[:, 16:]
            ia, ib = i3[:, :16], i3[:, 16:]
            sf = (ka > kb) | ((ka == kb) & (ia < ib))
            keys[b] = jnp.where(sf, ka, kb).reshape(nrows, C)
            idxs[b] = jnp.where(sf, ia, ib).reshape(nrows, C)
        rows = nrows
        for q in range(10, -1, -1):
            for b in range(nb):
                keys[b], idxs[b] = _cmpex(keys[b], idxs[b], npos, 10, q)
    for b in range(nb):
        idx_out[b] = idxs[b] + b * N_ANCHORS
        lidx_out[b] = idxs[b]
        key_out[b] = keys[b]


def _topk_sort(obj3):
    b = obj3.shape[0]
    return pl.pallas_call(
        _sort_body,
        out_shape=[jax.ShapeDtypeStruct((b, KR, C), jnp.int32),
                   jax.ShapeDtypeStruct((b, KR, C), jnp.int32),
                   jax.ShapeDtypeStruct((b, KR, C), jnp.float32)],
    )(obj3)


def _sc_gather(table, gidx):
    nrows = gidx.shape[0]
    info = plsc.get_sparse_core_info()
    nw = info.num_cores * info.num_subcores
    per = nrows // nw
    mesh = plsc.VectorSubcoreMesh(core_axis_name="c", subcore_axis_name="s")

    @functools.partial(
        pl.kernel,
        out_type=jax.ShapeDtypeStruct((nrows, TBL_W), jnp.float32),
        mesh=mesh,
        scratch_types=[pltpu.VMEM((per,), jnp.int32),
                       pltpu.VMEM((per, TBL_W), jnp.float32),
                       pltpu.SemaphoreType.DMA],
        compiler_params=pltpu.CompilerParams(use_tc_tiling_on_sc=False),
    )
    def k(table_hbm, idx_hbm, out_hbm, idx_v, rows_v, sem):
        wid = lax.axis_index("s") * info.num_cores + lax.axis_index("c")
        base = wid * per
        pltpu.sync_copy(idx_hbm.at[pl.ds(base, per)], idx_v)
        pltpu.async_copy(table_hbm.at[idx_v], rows_v, sem).wait()
        pltpu.sync_copy(rows_v, out_hbm.at[pl.ds(base, per)])

    return k(table, gidx)


def _decode(dx, dy, dw, dh, ax1, ay1, ax2, ay2):
    w = ax2 - ax1
    h = ay2 - ay1
    cx = ax1 + 0.5 * w
    cy = ay1 + 0.5 * h
    dwc = jnp.minimum(dw, BBOX_CLIP)
    dhc = jnp.minimum(dh, BBOX_CLIP)
    pcx = dx * w + cx
    pcy = dy * h + cy
    pw = jnp.exp(dwc) * w
    ph = jnp.exp(dhc) * h
    x1 = jnp.clip(pcx - 0.5 * pw, 0.0, IMG)
    y1 = jnp.clip(pcy - 0.5 * ph, 0.0, IMG)
    x2 = jnp.clip(pcx + 0.5 * pw, 0.0, IMG)
    y2 = jnp.clip(pcy + 0.5 * ph, 0.0, IMG)
    return x1, y1, x2, y2


RB = 256


def _build_m(b, msc, x1c, y1c, x2c, y2c, x1r, y1r, x2r, y2r,
             area_c, area_r, posc, posr, finite_c):
    for rb in range(K2 // RB):
        lo = rb * RB
        ltx = jnp.maximum(x1c[lo:lo + RB], x1r[:, lo:])
        lty = jnp.maximum(y1c[lo:lo + RB], y1r[:, lo:])
        rbx = jnp.minimum(x2c[lo:lo + RB], x2r[:, lo:])
        rby = jnp.minimum(y2c[lo:lo + RB], y2r[:, lo:])
        iw = jnp.maximum(rbx - ltx, 0.0)
        ih = jnp.maximum(rby - lty, 0.0)
        inter = iw * ih
        union = area_c[lo:lo + RB] + area_r[:, lo:] - inter
        iou = inter / jnp.maximum(union, 1e-9)
        m = ((iou > NMS_THRESH) & (posc[lo:lo + RB] < posr[:, lo:])
             & finite_c[lo:lo + RB])
        msc[b, lo:lo + RB, lo:] = m.astype(jnp.bfloat16)


def _nms_body(keyr_ref, g_ref,
              bo, so,
              msc):
    nb = keyr_ref.shape[0]
    posr = lax.broadcasted_iota(jnp.int32, (1, K2), 1)
    posc = lax.broadcasted_iota(jnp.int32, (K2, 1), 0)
    rowsd, finrs, keyrs = [], [], []
    for b in range(nb):
        keyr = keyr_ref[b]
        gb = g_ref[b * K2:(b + 1) * K2, :]
        gt = jnp.transpose(gb)
        cf = [gb[:, c:c + 1] for c in range(8)]
        rf = [gt[c:c + 1, :] for c in range(8)]
        x1r, y1r, x2r, y2r = _decode(*rf)
        x1c, y1c, x2c, y2c = _decode(*cf)
        finite_r = ((x2r - x1r >= MIN_SIZE) & (y2r - y1r >= MIN_SIZE)
                    & (posr < PRE_NMS))
        finite_c = ((x2c - x1c >= MIN_SIZE) & (y2c - y1c >= MIN_SIZE)
                    & (posc < PRE_NMS))
        area_r = jnp.maximum(x2r - x1r, 0.0) * jnp.maximum(y2r - y1r, 0.0)
        area_c = jnp.maximum(x2c - x1c, 0.0) * jnp.maximum(y2c - y1c, 0.0)
        _build_m(b, msc, x1c, y1c, x2c, y2c, x1r, y1r, x2r, y2r,
                 area_c, area_r, posc, posr, finite_c)
        rowsd.append((x1r, y1r, x2r, y2r))
        finrs.append(finite_r)
        keyrs.append(keyr)

    keep0 = jnp.concatenate(
        [jnp.broadcast_to(f.astype(jnp.float32), (8, K2)) for f in finrs],
        axis=0)

    def cond(st):
        it, changed, _ = st
        return changed & (it < K2)

    def body(st):
        it, _, keep = st
        kb = keep.astype(jnp.bfloat16)
        sups = []
        for b in range(nb):
            kbb = kb[8 * b:8 * b + 8]
            sups.append(jnp.concatenate(
                [jnp.dot(kbb[:, :cb * RB + RB],
                         msc[b, 0:cb * RB + RB, cb * RB:cb * RB + RB],
                         preferred_element_type=jnp.float32)
                 for cb in range(K2 // RB)], axis=1))
        sup = jnp.concatenate(sups, axis=0)
        keepn = jnp.where(sup > 0.5, 0.0, keep0)
        return it + 1, jnp.any(keepn != keep), keepn

    _, _, keep = lax.while_loop(cond, body,
                                (jnp.int32(0), jnp.bool_(True), keep0))

    p256r = lax.broadcasted_iota(jnp.int32, (1, RB), 1)
    p256c = lax.broadcasted_iota(jnp.int32, (RB, 1), 0)
    tri = (p256c <= p256r).astype(jnp.bfloat16)
    rr = lax.broadcasted_iota(jnp.int32, (OPAD, K2), 0) + 1
    for b in range(nb):
        kf = keep[8 * b:8 * b + 1]
        kb = keep[8 * b:8 * b + 8].astype(jnp.bfloat16)
        blocks = []
        off = jnp.float32(0.0)
        for cb in range(K2 // RB):
            lo = cb * RB
            rb_ = jnp.dot(kb[:, lo:lo + RB], tri,
                          preferred_element_type=jnp.float32) + off
            blocks.append(rb_)
            off = off + jnp.sum(kf[:, lo:lo + RB])
        rank = jnp.concatenate(blocks, axis=1)[0:1]
        rank_i = rank.astype(jnp.int32)
        P = ((rank_i == rr) & (kf > 0.5)).astype(jnp.float32)
        x1r, y1r, x2r, y2r = rowsd[b]
        score_r = jnp.where(finrs[b], keyrs[b], 0.0)
        bo[b] = jnp.concatenate(
            [jnp.sum(P * v, axis=1, keepdims=True)
             for v in (x1r, y1r, x2r, y2r)], axis=1)
        so[b] = jnp.sum(P * score_r, axis=1, keepdims=True)


def _nms_stage(keyr, gath):
    b = keyr.shape[0]
    return pl.pallas_call(
        _nms_body,
        out_shape=[jax.ShapeDtypeStruct((b, OPAD, 4), jnp.float32),
                   jax.ShapeDtypeStruct((b, OPAD, 1), jnp.float32)],
        scratch_shapes=[pltpu.VMEM((b, K2, K2), jnp.bfloat16)],
        compiler_params=pltpu.CompilerParams(
            vmem_limit_bytes=120 * 1024 * 1024),
    )(keyr, gath)


def kernel(objectness, pred_bbox_deltas, anchors):
    b = objectness.shape[0]
    n = anchors.shape[0]

    obj3 = jnp.full((b, NPAD), -jnp.inf, jnp.float32)
    obj3 = obj3.at[:, :n].set(objectness).reshape(b, R, C)
    sorted_idx, sorted_lidx, sorted_key = _topk_sort(obj3)

    del sorted_lidx
    anch_b = jnp.broadcast_to(anchors[None], (b, n, 4)).reshape(b * n, 4)
    table = jnp.concatenate(
        [pred_bbox_deltas.reshape(b * n, 4), anch_b,
         jnp.zeros((b * n, TBL_W - 8), jnp.float32)], axis=1)
    gath = _sc_gather(table, sorted_idx.reshape(b * K2))

    keyr = sorted_key.reshape(b, 1, K2)
    bo, so = _nms_stage(keyr, gath)

    boxes = bo[:, :POST_NMS, :]
    scores = so[:, :POST_NMS, 0]
    return boxes, scores


# speedup vs baseline: 1.3543x; 1.0052x over previous
import functools
import math

import jax
import jax.numpy as jnp
from jax import lax
from jax.experimental import pallas as pl
from jax.experimental.pallas import tpu as pltpu
from jax.experimental.pallas import tpu_sc as plsc

N_ANCHORS = 20000
NPAD = 32768
C = 128
R = NPAD // C
K2 = 2048
KR = K2 // C
PRE_NMS = 2000
POST_NMS = 1000
OPAD = 1024
NMS_THRESH = 0.7
MIN_SIZE = 1e-3
IMG = 800.0
BBOX_CLIP = float(math.log(1000.0 / 16.0))
TBL_W = 16


def _xshuf(x, d):
    if d < C:
        bit = (lax.broadcasted_iota(jnp.int32, x.shape, 1) & d) != 0
        return jnp.where(bit, pltpu.roll(x, d, axis=1),
                         pltpu.roll(x, C - d, axis=1))
    m = d // C
    bit = (lax.broadcasted_iota(jnp.int32, x.shape, 0) & m) != 0
    return jnp.where(bit, pltpu.roll(x, m, axis=0),
                     pltpu.roll(x, x.shape[0] - m, axis=0))


def _cmpex(key, idx, pos, p, q):
    d = 1 << q
    pk = _xshuf(key, d)
    pi = _xshuf(idx, d)
    self_first = (key > pk) | ((key == pk) & (idx < pi))
    run_up = (pos & (2 << p)) != 0
    hi_half = (pos & d) != 0
    take = self_first ^ run_up ^ hi_half
    return jnp.where(take, key, pk), jnp.where(take, idx, pi)


def _posmat(rows):
    return (lax.broadcasted_iota(jnp.int32, (rows, C), 0) * C
            + lax.broadcasted_iota(jnp.int32, (rows, C), 1))


def _sort_body(obj_ref, idx_out, lidx_out, key_out):
    nb = obj_ref.shape[0]
    pos = _posmat(R)
    keys = [obj_ref[b] for b in range(nb)]
    idxs = [pos] * nb
    for p in range(11):
        for q in range(p, -1, -1):
            for b in range(nb):
                keys[b], idxs[b] = _cmpex(keys[b], idxs[b], pos, p, q)
    rows = R
    while rows > KR:
        g = rows // 32
        nrows = rows // 2
        npos = _posmat(nrows)
        for b in range(nb):
            k3 = keys[b].reshape(g, 32, C)
            i3 = idxs[b].reshape(g, 32, C)
            ka, kb = k3[:, :16], k3[:, 16:]
            ia, ib = i3[:, :16], i3[:, 16:]
            sf = (ka > kb) | ((ka == kb) & (ia < ib))
            keys[b] = jnp.where(sf, ka, kb).reshape(nrows, C)
            idxs[b] = jnp.where(sf, ia, ib).reshape(nrows, C)
        rows = nrows
        for q in range(10, -1, -1):
            for b in range(nb):
                keys[b], idxs[b] = _cmpex(keys[b], idxs[b], npos, 10, q)
    for b in range(nb):
        idx_out[b] = idxs[b] + b * N_ANCHORS
        lidx_out[b] = idxs[b]
        key_out[b] = keys[b]


def _topk_sort(obj3):
    b = obj3.shape[0]
    return pl.pallas_call(
        _sort_body,
        out_shape=[jax.ShapeDtypeStruct((b, KR, C), jnp.int32),
                   jax.ShapeDtypeStruct((b, KR, C), jnp.int32),
                   jax.ShapeDtypeStruct((b, KR, C), jnp.float32)],
    )(obj3)


def _sc_gather(table, gidx):
    nrows = gidx.shape[0]
    info = plsc.get_sparse_core_info()
    nw = info.num_cores * info.num_subcores
    per = nrows // nw
    mesh = plsc.VectorSubcoreMesh(core_axis_name="c", subcore_axis_name="s")

    @functools.partial(
        pl.kernel,
        out_type=jax.ShapeDtypeStruct((nrows, TBL_W), jnp.float32),
        mesh=mesh,
        scratch_types=[pltpu.VMEM((per,), jnp.int32),
                       pltpu.VMEM((per, TBL_W), jnp.float32),
                       pltpu.SemaphoreType.DMA],
        compiler_params=pltpu.CompilerParams(use_tc_tiling_on_sc=False),
    )
    def k(table_hbm, idx_hbm, out_hbm, idx_v, rows_v, sem):
        wid = lax.axis_index("s") * info.num_cores + lax.axis_index("c")
        base = wid * per
        pltpu.sync_copy(idx_hbm.at[pl.ds(base, per)], idx_v)
        pltpu.async_copy(table_hbm.at[idx_v], rows_v, sem).wait()
        pltpu.sync_copy(rows_v, out_hbm.at[pl.ds(base, per)])

    return k(table, gidx)


def _decode(dx, dy, dw, dh, ax1, ay1, ax2, ay2):
    w = ax2 - ax1
    h = ay2 - ay1
    cx = ax1 + 0.5 * w
    cy = ay1 + 0.5 * h
    dwc = jnp.minimum(dw, BBOX_CLIP)
    dhc = jnp.minimum(dh, BBOX_CLIP)
    pcx = dx * w + cx
    pcy = dy * h + cy
    pw = jnp.exp(dwc) * w
    ph = jnp.exp(dhc) * h
    x1 = jnp.clip(pcx - 0.5 * pw, 0.0, IMG)
    y1 = jnp.clip(pcy - 0.5 * ph, 0.0, IMG)
    x2 = jnp.clip(pcx + 0.5 * pw, 0.0, IMG)
    y2 = jnp.clip(pcy + 0.5 * ph, 0.0, IMG)
    return x1, y1, x2, y2


RB = 256


def _build_m(b, msc, x1c, y1c, x2c, y2c, x1r, y1r, x2r, y2r,
             area_c, area_r, posc, posr, finite_c):
    sup0 = jnp.zeros((1, K2), jnp.bool_)
    for rb in range(K2 // RB):
        lo = rb * RB
        ltx = jnp.maximum(x1c[lo:lo + RB], x1r[:, lo:])
        lty = jnp.maximum(y1c[lo:lo + RB], y1r[:, lo:])
        rbx = jnp.minimum(x2c[lo:lo + RB], x2r[:, lo:])
        rby = jnp.minimum(y2c[lo:lo + RB], y2r[:, lo:])
        iw = jnp.maximum(rbx - ltx, 0.0)
        ih = jnp.maximum(rby - lty, 0.0)
        inter = iw * ih
        union = area_c[lo:lo + RB] + area_r[:, lo:] - inter
        iou = inter / jnp.maximum(union, 1e-9)
        m = ((iou > NMS_THRESH) & (posc[lo:lo + RB] < posr[:, lo:])
             & finite_c[lo:lo + RB])
        msc[b, lo:lo + RB, lo:] = m.astype(jnp.bfloat16)
        blk = jnp.any(m, axis=0, keepdims=True)
        if lo:
            blk = jnp.concatenate(
                [jnp.zeros((1, lo), jnp.bool_), blk], axis=1)
        sup0 = sup0 | blk
    return sup0


def _nms_body(keyr_ref, g_ref,
              bo, so,
              msc):
    nb = keyr_ref.shape[0]
    posr = lax.broadcasted_iota(jnp.int32, (1, K2), 1)
    posc = lax.broadcasted_iota(jnp.int32, (K2, 1), 0)
    rowsd, finrs, keyrs = [], [], []
    for b in range(nb):
        keyr = keyr_ref[b]
        gb = g_ref[b * K2:(b + 1) * K2, :]
        gt = jnp.transpose(gb)
        cf = [gb[:, c:c + 1] for c in range(8)]
        rf = [gt[c:c + 1, :] for c in range(8)]
        x1r, y1r, x2r, y2r = _decode(*rf)
        x1c, y1c, x2c, y2c = _decode(*cf)
        finite_r = ((x2r - x1r >= MIN_SIZE) & (y2r - y1r >= MIN_SIZE)
                    & (posr < PRE_NMS))
        finite_c = ((x2c - x1c >= MIN_SIZE) & (y2c - y1c >= MIN_SIZE)
                    & (posc < PRE_NMS))
        area_r = jnp.maximum(x2r - x1r, 0.0) * jnp.maximum(y2r - y1r, 0.0)
        area_c = jnp.maximum(x2c - x1c, 0.0) * jnp.maximum(y2c - y1c, 0.0)
        sup0 = _build_m(b, msc, x1c, y1c, x2c, y2c, x1r, y1r, x2r, y2r,
                        area_c, area_r, posc, posr, finite_c)
        rowsd.append((x1r, y1r, x2r, y2r, sup0))
        finrs.append(finite_r)
        keyrs.append(keyr)

    keep0 = jnp.concatenate(
        [jnp.broadcast_to(f.astype(jnp.float32), (8, K2)) for f in finrs],
        axis=0)
    keep1 = jnp.concatenate(
        [jnp.broadcast_to(
            jnp.where(rowsd[b][4], 0.0, finrs[b].astype(jnp.float32)),
            (8, K2)) for b in range(nb)], axis=0)

    def cond(st):
        it, changed, _ = st
        return changed & (it < K2)

    def body(st):
        it, _, keep = st
        kb = keep.astype(jnp.bfloat16)
        sups = []
        for b in range(nb):
            kbb = kb[8 * b:8 * b + 8]
            sups.append(jnp.concatenate(
                [jnp.dot(kbb[:, :cb * RB + RB],
                         msc[b, 0:cb * RB + RB, cb * RB:cb * RB + RB],
                         preferred_element_type=jnp.float32)
                 for cb in range(K2 // RB)], axis=1))
        sup = jnp.concatenate(sups, axis=0)
        keepn = jnp.where(sup > 0.5, 0.0, keep0)
        changed = jnp.bool_(False)
        for b in range(nb):
            changed = changed | jnp.any(
                keepn[8 * b:8 * b + 1] != keep[8 * b:8 * b + 1])
        return it + 1, changed, keepn

    _, _, keep = lax.while_loop(cond, body,
                                (jnp.int32(1), jnp.bool_(True), keep1))

    p256r = lax.broadcasted_iota(jnp.int32, (1, RB), 1)
    p256c = lax.broadcasted_iota(jnp.int32, (RB, 1), 0)
    tri = (p256c <= p256r).astype(jnp.bfloat16)
    rr = lax.broadcasted_iota(jnp.int32, (OPAD, K2), 0) + 1
    for b in range(nb):
        kf = keep[8 * b:8 * b + 1]
        kb = keep[8 * b:8 * b + 8].astype(jnp.bfloat16)
        blocks = []
        off = jnp.float32(0.0)
        for cb in range(K2 // RB):
            lo = cb * RB
            rb_ = jnp.dot(kb[:, lo:lo + RB], tri,
                          preferred_element_type=jnp.float32) + off
            blocks.append(rb_)
            off = off + jnp.sum(kf[:, lo:lo + RB])
        rank = jnp.concatenate(blocks, axis=1)[0:1]
        rank_i = rank.astype(jnp.int32)
        P = ((rank_i == rr) & (kf > 0.5)).astype(jnp.float32)
        x1r, y1r, x2r, y2r = rowsd[b][:4]
        score_r = jnp.where(finrs[b], keyrs[b], 0.0)
        bo[b] = jnp.concatenate(
            [jnp.sum(P * v, axis=1, keepdims=True)
             for v in (x1r, y1r, x2r, y2r)], axis=1)
        so[b] = jnp.sum(P * score_r, axis=1, keepdims=True)


def _nms_stage(keyr, gath):
    b = keyr.shape[0]
    return pl.pallas_call(
        _nms_body,
        out_shape=[jax.ShapeDtypeStruct((b, OPAD, 4), jnp.float32),
                   jax.ShapeDtypeStruct((b, OPAD, 1), jnp.float32)],
        scratch_shapes=[pltpu.VMEM((b, K2, K2), jnp.bfloat16)],
        compiler_params=pltpu.CompilerParams(
            vmem_limit_bytes=120 * 1024 * 1024),
    )(keyr, gath)


def kernel(objectness, pred_bbox_deltas, anchors):
    b = objectness.shape[0]
    n = anchors.shape[0]

    obj3 = jnp.full((b, NPAD), -jnp.inf, jnp.float32)
    obj3 = obj3.at[:, :n].set(objectness).reshape(b, R, C)
    sorted_idx, sorted_lidx, sorted_key = _topk_sort(obj3)

    del sorted_lidx
    anch_b = jnp.broadcast_to(anchors[None], (b, n, 4)).reshape(b * n, 4)
    table = jnp.concatenate(
        [pred_bbox_deltas.reshape(b * n, 4), anch_b,
         jnp.zeros((b * n, TBL_W - 8), jnp.float32)], axis=1)
    gath = _sc_gather(table, sorted_idx.reshape(b * K2))

    keyr = sorted_key.reshape(b, 1, K2)
    bo, so = _nms_stage(keyr, gath)

    boxes = bo[:, :POST_NMS, :]
    scores = so[:, :POST_NMS, 0]
    return boxes, scores


# speedup vs baseline: 1.3689x; 1.0108x over previous
import functools
import math

import jax
import jax.numpy as jnp
from jax import lax
from jax.experimental import pallas as pl
from jax.experimental.pallas import tpu as pltpu
from jax.experimental.pallas import tpu_sc as plsc

N_ANCHORS = 20000
NPAD = 32768
C = 128
R = NPAD // C
K2 = 2048
KR = K2 // C
PRE_NMS = 2000
POST_NMS = 1000
OPAD = 1024
NMS_THRESH = 0.7
MIN_SIZE = 1e-3
IMG = 800.0
BBOX_CLIP = float(math.log(1000.0 / 16.0))
TBL_W = 16


def _xshuf(x, d):
    if d < C:
        bit = (lax.broadcasted_iota(jnp.int32, x.shape, 1) & d) != 0
        return jnp.where(bit, pltpu.roll(x, d, axis=1),
                         pltpu.roll(x, C - d, axis=1))
    m = d // C
    bit = (lax.broadcasted_iota(jnp.int32, x.shape, 0) & m) != 0
    return jnp.where(bit, pltpu.roll(x, m, axis=0),
                     pltpu.roll(x, x.shape[0] - m, axis=0))


def _cmpex(key, idx, pos, p, q):
    d = 1 << q
    pk = _xshuf(key, d)
    pi = _xshuf(idx, d)
    self_first = (key > pk) | ((key == pk) & (idx < pi))
    run_up = (pos & (2 << p)) != 0
    hi_half = (pos & d) != 0
    take = self_first ^ run_up ^ hi_half
    return jnp.where(take, key, pk), jnp.where(take, idx, pi)


def _posmat(rows):
    return (lax.broadcasted_iota(jnp.int32, (rows, C), 0) * C
            + lax.broadcasted_iota(jnp.int32, (rows, C), 1))


def _sort_body(obj_ref, idx_out, lidx_out, key_out):
    nb = obj_ref.shape[0]
    pos = _posmat(R)
    keys = [obj_ref[b] for b in range(nb)]
    idxs = [pos] * nb
    for p in range(11):
        for q in range(p, -1, -1):
            for b in range(nb):
                keys[b], idxs[b] = _cmpex(keys[b], idxs[b], pos, p, q)
    rows = R
    while rows > KR:
        g = rows // 32
        nrows = rows // 2
        npos = _posmat(nrows)
        for b in range(nb):
            k3 = keys[b].reshape(g, 32, C)
            i3 = idxs[b].reshape(g, 32, C)
            ka, kb = k3[:, :16], k3[:, 16:]
            ia, ib = i3[:, :16], i3[:, 16:]
            sf = (ka > kb) | ((ka == kb) & (ia < ib))
            keys[b] = jnp.where(sf, ka, kb).reshape(nrows, C)
            idxs[b] = jnp.where(sf, ia, ib).reshape(nrows, C)
        rows = nrows
        for q in range(10, -1, -1):
            for b in range(nb):
                keys[b], idxs[b] = _cmpex(keys[b], idxs[b], npos, 10, q)
    for b in range(nb):
        idx_out[b] = idxs[b] + b * N_ANCHORS
        lidx_out[b] = idxs[b]
        key_out[b] = keys[b]


def _topk_sort(obj3):
    b = obj3.shape[0]
    return pl.pallas_call(
        _sort_body,
        out_shape=[jax.ShapeDtypeStruct((b, KR, C), jnp.int32),
                   jax.ShapeDtypeStruct((b, KR, C), jnp.int32),
                   jax.ShapeDtypeStruct((b, KR, C), jnp.float32)],
    )(obj3)


def _sc_gather(table, gidx):
    nrows = gidx.shape[0]
    info = plsc.get_sparse_core_info()
    ncores = 1
    nw = ncores * info.num_subcores
    per = nrows // nw
    mesh = plsc.VectorSubcoreMesh(core_axis_name="c", subcore_axis_name="s",
                                  num_cores=ncores)

    @functools.partial(
        pl.kernel,
        out_type=jax.ShapeDtypeStruct((nrows, TBL_W), jnp.float32),
        mesh=mesh,
        scratch_types=[pltpu.VMEM((per,), jnp.int32),
                       pltpu.VMEM((per, TBL_W), jnp.float32),
                       pltpu.SemaphoreType.DMA],
        compiler_params=pltpu.CompilerParams(use_tc_tiling_on_sc=False),
    )
    def k(table_hbm, idx_hbm, out_hbm, idx_v, rows_v, sem):
        wid = lax.axis_index("s") * ncores + lax.axis_index("c")
        base = wid * per
        pltpu.sync_copy(idx_hbm.at[pl.ds(base, per)], idx_v)
        pltpu.async_copy(table_hbm.at[idx_v], rows_v, sem).wait()
        pltpu.sync_copy(rows_v, out_hbm.at[pl.ds(base, per)])

    return k(table, gidx)


def _decode(dx, dy, dw, dh, ax1, ay1, ax2, ay2):
    w = ax2 - ax1
    h = ay2 - ay1
    cx = ax1 + 0.5 * w
    cy = ay1 + 0.5 * h
    dwc = jnp.minimum(dw, BBOX_CLIP)
    dhc = jnp.minimum(dh, BBOX_CLIP)
    pcx = dx * w + cx
    pcy = dy * h + cy
    pw = jnp.exp(dwc) * w
    ph = jnp.exp(dhc) * h
    x1 = jnp.clip(pcx - 0.5 * pw, 0.0, IMG)
    y1 = jnp.clip(pcy - 0.5 * ph, 0.0, IMG)
    x2 = jnp.clip(pcx + 0.5 * pw, 0.0, IMG)
    y2 = jnp.clip(pcy + 0.5 * ph, 0.0, IMG)
    return x1, y1, x2, y2


RB = 256


def _build_m(b, msc, x1c, y1c, x2c, y2c, x1r, y1r, x2r, y2r,
             area_c, area_r, posc, posr, finite_c):
    sup0 = jnp.zeros((1, K2), jnp.bool_)
    for rb in range(K2 // RB):
        lo = rb * RB
        ltx = jnp.maximum(x1c[lo:lo + RB], x1r[:, lo:])
        lty = jnp.maximum(y1c[lo:lo + RB], y1r[:, lo:])
        rbx = jnp.minimum(x2c[lo:lo + RB], x2r[:, lo:])
        rby = jnp.minimum(y2c[lo:lo + RB], y2r[:, lo:])
        iw = jnp.maximum(rbx - ltx, 0.0)
        ih = jnp.maximum(rby - lty, 0.0)
        inter = iw * ih
        union = area_c[lo:lo + RB] + area_r[:, lo:] - inter
        iou = inter / jnp.maximum(union, 1e-9)
        m = ((iou > NMS_THRESH) & (posc[lo:lo + RB] < posr[:, lo:])
             & finite_c[lo:lo + RB])
        msc[b, lo:lo + RB, lo:] = m.astype(jnp.bfloat16)
        blk = jnp.any(m, axis=0, keepdims=True)
        if lo:
            blk = jnp.concatenate(
                [jnp.zeros((1, lo), jnp.bool_), blk], axis=1)
        sup0 = sup0 | blk
    return sup0


def _nms_body(keyr_ref, g_ref,
              bo, so,
              msc):
    nb = keyr_ref.shape[0]
    posr = lax.broadcasted_iota(jnp.int32, (1, K2), 1)
    posc = lax.broadcasted_iota(jnp.int32, (K2, 1), 0)
    rowsd, finrs, keyrs = [], [], []
    for b in range(nb):
        keyr = keyr_ref[b]
        gb = g_ref[b * K2:(b + 1) * K2, :]
        gt = jnp.transpose(gb)
        cf = [gb[:, c:c + 1] for c in range(8)]
        rf = [gt[c:c + 1, :] for c in range(8)]
        x1r, y1r, x2r, y2r = _decode(*rf)
        x1c, y1c, x2c, y2c = _decode(*cf)
        finite_r = ((x2r - x1r >= MIN_SIZE) & (y2r - y1r >= MIN_SIZE)
                    & (posr < PRE_NMS))
        finite_c = ((x2c - x1c >= MIN_SIZE) & (y2c - y1c >= MIN_SIZE)
                    & (posc < PRE_NMS))
        area_r = jnp.maximum(x2r - x1r, 0.0) * jnp.maximum(y2r - y1r, 0.0)
        area_c = jnp.maximum(x2c - x1c, 0.0) * jnp.maximum(y2c - y1c, 0.0)
        sup0 = _build_m(b, msc, x1c, y1c, x2c, y2c, x1r, y1r, x2r, y2r,
                        area_c, area_r, posc, posr, finite_c)
        rowsd.append((x1r, y1r, x2r, y2r, sup0))
        finrs.append(finite_r)
        keyrs.append(keyr)

    keep0 = jnp.concatenate(
        [jnp.broadcast_to(f.astype(jnp.float32), (8, K2)) for f in finrs],
        axis=0)
    keep1 = jnp.concatenate(
        [jnp.broadcast_to(
            jnp.where(rowsd[b][4], 0.0, finrs[b].astype(jnp.float32)),
            (8, K2)) for b in range(nb)], axis=0)

    def cond(st):
        it, changed, _ = st
        return changed & (it < K2)

    def body(st):
        it, _, keep = st
        kb = keep.astype(jnp.bfloat16)
        sups = []
        for b in range(nb):
            kbb = kb[8 * b:8 * b + 8]
            sups.append(jnp.concatenate(
                [jnp.dot(kbb[:, :cb * RB + RB],
                         msc[b, 0:cb * RB + RB, cb * RB:cb * RB + RB],
                         preferred_element_type=jnp.float32)
                 for cb in range(K2 // RB)], axis=1))
        sup = jnp.concatenate(sups, axis=0)
        keepn = jnp.where(sup > 0.5, 0.0, keep0)
        changed = jnp.bool_(False)
        for b in range(nb):
            changed = changed | jnp.any(
                keepn[8 * b:8 * b + 1] != keep[8 * b:8 * b + 1])
        return it + 1, changed, keepn

    _, _, keep = lax.while_loop(cond, body,
                                (jnp.int32(1), jnp.bool_(True), keep1))

    p256r = lax.broadcasted_iota(jnp.int32, (1, RB), 1)
    p256c = lax.broadcasted_iota(jnp.int32, (RB, 1), 0)
    tri = (p256c <= p256r).astype(jnp.bfloat16)
    rr = lax.broadcasted_iota(jnp.int32, (OPAD, K2), 0) + 1
    for b in range(nb):
        kf = keep[8 * b:8 * b + 1]
        kb = keep[8 * b:8 * b + 8].astype(jnp.bfloat16)
        blocks = []
        off = jnp.float32(0.0)
        for cb in range(K2 // RB):
            lo = cb * RB
            rb_ = jnp.dot(kb[:, lo:lo + RB], tri,
                          preferred_element_type=jnp.float32) + off
            blocks.append(rb_)
            off = off + jnp.sum(kf[:, lo:lo + RB])
        rank = jnp.concatenate(blocks, axis=1)[0:1]
        rank_i = rank.astype(jnp.int32)
        P = ((rank_i == rr) & (kf > 0.5)).astype(jnp.float32)
        x1r, y1r, x2r, y2r = rowsd[b][:4]
        score_r = jnp.where(finrs[b], keyrs[b], 0.0)
        bo[b] = jnp.concatenate(
            [jnp.sum(P * v, axis=1, keepdims=True)
             for v in (x1r, y1r, x2r, y2r)], axis=1)
        so[b] = jnp.sum(P * score_r, axis=1, keepdims=True)


def _nms_stage(keyr, gath):
    b = keyr.shape[0]
    return pl.pallas_call(
        _nms_body,
        out_shape=[jax.ShapeDtypeStruct((b, OPAD, 4), jnp.float32),
                   jax.ShapeDtypeStruct((b, OPAD, 1), jnp.float32)],
        scratch_shapes=[pltpu.VMEM((b, K2, K2), jnp.bfloat16)],
        compiler_params=pltpu.CompilerParams(
            vmem_limit_bytes=120 * 1024 * 1024),
    )(keyr, gath)


def kernel(objectness, pred_bbox_deltas, anchors):
    b = objectness.shape[0]
    n = anchors.shape[0]

    obj3 = jnp.full((b, NPAD), -jnp.inf, jnp.float32)
    obj3 = obj3.at[:, :n].set(objectness).reshape(b, R, C)
    sorted_idx, sorted_lidx, sorted_key = _topk_sort(obj3)

    del sorted_lidx
    anch_b = jnp.broadcast_to(anchors[None], (b, n, 4)).reshape(b * n, 4)
    table = jnp.concatenate(
        [pred_bbox_deltas.reshape(b * n, 4), anch_b,
         jnp.zeros((b * n, TBL_W - 8), jnp.float32)], axis=1)
    gath = _sc_gather(table, sorted_idx.reshape(b * K2))

    keyr = sorted_key.reshape(b, 1, K2)
    bo, so = _nms_stage(keyr, gath)

    boxes = bo[:, :POST_NMS, :]
    scores = so[:, :POST_NMS, 0]
    return boxes, scores
